# Initial kernel scaffold; baseline (speedup 1.0000x reference)
import jax
import jax.numpy as jnp
from jax.experimental import pallas as pl


def kernel(coords, features, W1, W2, W3):
    raise NotImplementedError("write your pallas kernel here")


# speedup vs baseline: 5.1796x; 5.1796x over previous
import functools
import jax
import jax.numpy as jnp
from jax.experimental import pallas as pl
from jax.experimental.pallas import tpu as pltpu

_K = 10


def _knn_body(ct_ref, c_ref, out_ref, d_scr):
    TM = ct_ref.shape[0]
    NP = c_ref.shape[1]
    CH = 256
    NCH = NP // CH
    INF = jnp.float32(3e38)
    BIGI = jnp.int32(2**30)

    xi = ct_ref[:, 0:1]
    yi = ct_ref[:, 1:2]
    zi = ct_ref[:, 2:3]
    sqi = xi * xi + yi * yi + zi * zi

    xib = xi.astype(jnp.bfloat16).astype(jnp.float32)
    yib = yi.astype(jnp.bfloat16).astype(jnp.float32)
    zib = zi.astype(jnp.bfloat16).astype(jnp.float32)

    def pass1(c, carry):
        sl = pl.ds(c * CH, CH)
        xj = c_ref[0:1, sl]
        yj = c_ref[1:2, sl]
        zj = c_ref[2:3, sl]
        sqj = xj * xj + yj * yj + zj * zj
        xjb = xj.astype(jnp.bfloat16).astype(jnp.float32)
        yjb = yj.astype(jnp.bfloat16).astype(jnp.float32)
        zjb = zj.astype(jnp.bfloat16).astype(jnp.float32)
        d = sqi + sqj - 2.0 * (xib * xjb + yib * yjb + zib * zjb)
        d_scr[:, sl] = d
        return carry

    jax.lax.fori_loop(0, NCH, pass1, 0)

    lane_out = jax.lax.broadcasted_iota(jnp.int32, (TM, 128), 1)

    def ext_body(t, carry):
        acc, prev_idx = carry

        def chunk_body(c, ch_carry):
            bval, bidx = ch_carry
            sl = pl.ds(c * CH, CH)
            d = d_scr[:, sl]
            gcol = jax.lax.broadcasted_iota(jnp.int32, (TM, CH), 1) + c * CH
            dm = jnp.where(gcol == prev_idx, INF, d)
            d_scr[:, sl] = dm
            cmin = jnp.min(dm, axis=1, keepdims=True)
            cidx = jnp.min(jnp.where(dm == cmin, gcol, BIGI), axis=1,
                           keepdims=True)
            better = cmin < bval
            return (jnp.where(better, cmin, bval),
                    jnp.where(better, cidx, bidx))

        bval0 = jnp.full((TM, 1), INF, jnp.float32)
        bidx0 = jnp.full((TM, 1), BIGI, jnp.int32)
        _, bidx = jax.lax.fori_loop(0, NCH, chunk_body, (bval0, bidx0))
        acc = jnp.where(lane_out == t, bidx, acc)
        return acc, bidx

    acc0 = jnp.zeros((TM, 128), jnp.int32)
    prev0 = jnp.full((TM, 1), -1, jnp.int32)
    acc, _ = jax.lax.fori_loop(0, _K + 1, ext_body, (acc0, prev0))
    out_ref[:, :] = acc


def _knn_idx(coords, tm=256):
    B, _, N = coords.shape
    NP = ((N + tm - 1) // tm) * tm
    if NP % 256:
        NP = ((NP + 255) // 256) * 256
    c = coords[0]
    c_pad = jnp.full((8, NP), 1e6, jnp.float32)
    c_pad = c_pad.at[:3, :N].set(c)
    c_pad = c_pad.at[3:, :].set(0.0)
    ct_pad = c_pad.T

    grid = (NP // tm,)
    idx_full = pl.pallas_call(
        _knn_body,
        grid=grid,
        in_specs=[
            pl.BlockSpec((tm, 8), lambda i: (i, 0)),
            pl.BlockSpec((8, NP), lambda i: (0, 0)),
        ],
        out_specs=pl.BlockSpec((tm, 128), lambda i: (i, 0)),
        out_shape=jax.ShapeDtypeStruct((NP, 128), jnp.int32),
        scratch_shapes=[pltpu.VMEM((tm, NP), jnp.float32)],
    )(ct_pad, c_pad)
    return idx_full[None, :N, 1:1 + _K]


def _graph_feature(feats, idx):
    neighbor = jax.vmap(lambda f, i: f[:, i])(feats, idx)
    center = jnp.broadcast_to(feats[:, :, :, None], neighbor.shape)
    return jnp.concatenate([center, neighbor - center], axis=1)


def _instance_norm(x, eps=1e-5):
    m = jnp.mean(x, axis=(2, 3), keepdims=True)
    v = jnp.mean((x - m) ** 2, axis=(2, 3), keepdims=True)
    return (x - m) / jnp.sqrt(v + eps)


def _conv1x1(W, x):
    return jnp.einsum('oc,bcnk->bonk', W, x)


def _leaky(x):
    return jnp.where(x >= 0, x, 0.2 * x)


def kernel(coords, features, W1, W2, W3):
    B, C, N = features.shape
    idx = _knn_idx(coords)
    x0 = features[:, :, :, None]
    x1 = _graph_feature(features, idx)
    x1 = _leaky(_instance_norm(_conv1x1(W1, x1)))
    x1 = jnp.max(x1, axis=-1, keepdims=True)
    x2 = _graph_feature(x1[..., 0], idx)
    x2 = _leaky(_instance_norm(_conv1x1(W2, x2)))
    x2 = jnp.max(x2, axis=-1, keepdims=True)
    x3 = jnp.concatenate([x0, x1, x2], axis=1)
    x3 = _leaky(_instance_norm(_conv1x1(W3, x3)))
    return x3.reshape(B, -1, N)


# speedup vs baseline: 6.6991x; 1.2934x over previous
import functools
import jax
import jax.numpy as jnp
from jax import lax
from jax.experimental import pallas as pl
from jax.experimental.pallas import tpu as pltpu
from jax.experimental.pallas import tpu_sc as plsc

_K = 10
_F32 = jnp.float32
_HI = jax.lax.Precision.HIGHEST


def _dot(a, b):
    return jax.lax.dot_general(a, b, (((1,), (0,)), ((), ())),
                               preferred_element_type=_F32, precision=_HI)


def _knn_body(ct_ref, c_ref, out_ref, d_scr):
    TM = ct_ref.shape[0]
    NP = c_ref.shape[1]
    CH = 256
    NCH = NP // CH
    INF = jnp.float32(3e38)
    BIGI = jnp.int32(2**30)

    xi = ct_ref[:, 0:1]
    yi = ct_ref[:, 1:2]
    zi = ct_ref[:, 2:3]
    sqi = xi * xi + yi * yi + zi * zi
    xib = xi.astype(jnp.bfloat16).astype(_F32)
    yib = yi.astype(jnp.bfloat16).astype(_F32)
    zib = zi.astype(jnp.bfloat16).astype(_F32)

    def pass1(c, carry):
        sl = pl.ds(c * CH, CH)
        xj = c_ref[0:1, sl]
        yj = c_ref[1:2, sl]
        zj = c_ref[2:3, sl]
        sqj = xj * xj + yj * yj + zj * zj
        xjb = xj.astype(jnp.bfloat16).astype(_F32)
        yjb = yj.astype(jnp.bfloat16).astype(_F32)
        zjb = zj.astype(jnp.bfloat16).astype(_F32)
        d = sqi + sqj - 2.0 * (xib * xjb + yib * yjb + zib * zjb)
        d_scr[:, sl] = d
        return carry

    jax.lax.fori_loop(0, NCH, pass1, 0)

    lane_out = jax.lax.broadcasted_iota(jnp.int32, (TM, 128), 1)

    def ext_body(t, carry):
        acc, prev_idx = carry

        def chunk_body(c, ch_carry):
            bval, bidx = ch_carry
            sl = pl.ds(c * CH, CH)
            d = d_scr[:, sl]
            gcol = jax.lax.broadcasted_iota(jnp.int32, (TM, CH), 1) + c * CH
            dm = jnp.where(gcol == prev_idx, INF, d)
            d_scr[:, sl] = dm
            cmin = jnp.min(dm, axis=1, keepdims=True)
            cidx = jnp.min(jnp.where(dm == cmin, gcol, BIGI), axis=1,
                           keepdims=True)
            better = cmin < bval
            return (jnp.where(better, cmin, bval),
                    jnp.where(better, cidx, bidx))

        bval0 = jnp.full((TM, 1), INF, jnp.float32)
        bidx0 = jnp.full((TM, 1), BIGI, jnp.int32)
        _, bidx = jax.lax.fori_loop(0, NCH, chunk_body, (bval0, bidx0))
        acc = jnp.where(lane_out == t, bidx, acc)
        return acc, bidx

    acc0 = jnp.zeros((TM, 128), jnp.int32)
    prev0 = jnp.full((TM, 1), -1, jnp.int32)
    acc, _ = jax.lax.fori_loop(0, _K + 1, ext_body, (acc0, prev0))
    out_ref[:, :] = acc


def _knn_pallas(coords, NP, tm=256):
    _, _, N = coords.shape
    c = coords[0]
    c_pad = jnp.full((8, NP), 1e6, jnp.float32)
    c_pad = c_pad.at[:3, :N].set(c)
    c_pad = c_pad.at[3:, :].set(0.0)
    ct_pad = c_pad.T

    return pl.pallas_call(
        _knn_body,
        grid=(NP // tm,),
        in_specs=[
            pl.BlockSpec((tm, 8), lambda i: (i, 0)),
            pl.BlockSpec((8, NP), lambda i: (0, 0)),
        ],
        out_specs=pl.BlockSpec((tm, 128), lambda i: (i, 0)),
        out_shape=jax.ShapeDtypeStruct((NP, 128), jnp.int32),
        scratch_shapes=[pltpu.VMEM((tm, NP), jnp.float32)],
    )(ct_pad, c_pad)


def _gr_body(bt_hbm, idx_hbm, out_hbm, idx_v, rows_v, s_v, m_v, q_v, sem,
             *, PPW, CP, GP, C):
    NF = CP // GP
    NCHK = PPW // CP
    wid = lax.axis_index("s") * 2 + lax.axis_index("c")
    base = wid * PPW

    def chunk(ch, carry):
        pbase = base + ch * CP
        pltpu.sync_copy(idx_hbm.at[pl.ds(pbase * _K, CP * _K)], idx_v)
        cps = [pltpu.async_copy(
                   bt_hbm.at[idx_v.at[pl.ds(f * GP * _K, GP * _K)]],
                   rows_v.at[pl.ds(f * GP * _K, GP * _K)], sem)
               for f in range(NF)]
        for cp_ in cps:
            cp_.wait()

        def point(p, c2):
            for c in range(C // 16):
                sl = pl.ds(c * 16, 16)
                r = rows_v[p * _K, sl]
                s = r
                m = r
                q = r * r
                for j in range(1, _K):
                    r = rows_v[p * _K + j, sl]
                    s = s + r
                    m = jnp.maximum(m, r)
                    q = q + r * r
                s_v[p, sl] = s
                m_v[p, sl] = m
                q_v[p, sl] = q
            return c2

        jax.lax.fori_loop(0, CP, point, 0)
        pltpu.sync_copy(s_v, out_hbm.at[0, pl.ds(pbase, CP)])
        pltpu.sync_copy(m_v, out_hbm.at[1, pl.ds(pbase, CP)])
        pltpu.sync_copy(q_v, out_hbm.at[2, pl.ds(pbase, CP)])
        return carry

    jax.lax.fori_loop(0, NCHK, chunk, 0)


def _gather_reduce(bt, idxf):
    NP, C = bt.shape
    NW = 32
    PPW = NP // NW
    CP = 16
    GP = 8
    mesh = plsc.VectorSubcoreMesh(core_axis_name="c", subcore_axis_name="s")
    body = functools.partial(_gr_body, PPW=PPW, CP=CP, GP=GP, C=C)
    f = functools.partial(
        pl.kernel, mesh=mesh,
        out_type=jax.ShapeDtypeStruct((3, NP, C), jnp.float32),
        scratch_types=[
            pltpu.VMEM((CP * _K,), jnp.int32),
            pltpu.VMEM((CP * _K, C), jnp.float32),
            pltpu.VMEM((CP, C), jnp.float32),
            pltpu.VMEM((CP, C), jnp.float32),
            pltpu.VMEM((CP, C), jnp.float32),
            pltpu.SemaphoreType.DMA,
        ])(body)
    return f(bt, idxf)


def _mm_body(x_ref, w_ref, *out_refs):
    x = x_ref[:, :]
    off = 0
    for o in out_refs:
        co = o.shape[1]
        o[:, :] = _dot(x, w_ref[:, pl.ds(off, co)])
        off += co


def _matmul(x, w, splits, tm=1024):
    NP, cin = x.shape
    cout = w.shape[1]
    return pl.pallas_call(
        _mm_body,
        grid=(NP // tm,),
        in_specs=[
            pl.BlockSpec((tm, cin), lambda i: (i, 0)),
            pl.BlockSpec((cin, cout), lambda i: (0, 0)),
        ],
        out_specs=[pl.BlockSpec((tm, co), lambda i: (i, 0)) for co in splits],
        out_shape=[jax.ShapeDtypeStruct((NP, co), jnp.float32)
                   for co in splits],
    )(x, w)


def _stats_body(a_ref, s_ref, q_ref, o_ref, *, N, tm):
    i = pl.program_id(0)
    C = a_ref.shape[1]
    row = i * tm + jax.lax.broadcasted_iota(jnp.int32, (tm, 1), 0)
    msk = (row < N).astype(_F32)
    a = a_ref[:, :] * msk
    s = s_ref[0, :, :] * msk
    q = q_ref[0, :, :] * msk
    part = jnp.concatenate([
        jnp.sum(a, 0, keepdims=True),
        jnp.sum(s, 0, keepdims=True),
        jnp.sum(q, 0, keepdims=True),
        jnp.sum(a * a, 0, keepdims=True),
        jnp.sum(a * s, 0, keepdims=True),
    ], axis=1)

    @pl.when(i == 0)
    def _():
        o_ref[:, :] = jnp.zeros_like(o_ref)

    o_ref[0:1, :] += part


def _stats(a, smq, N, tm=1024):
    NP, C = a.shape
    body = functools.partial(_stats_body, N=N, tm=tm)
    return pl.pallas_call(
        body,
        grid=(NP // tm,),
        in_specs=[
            pl.BlockSpec((tm, C), lambda i: (i, 0)),
            pl.BlockSpec((1, tm, C), lambda i: (0, i, 0)),
            pl.BlockSpec((1, tm, C), lambda i: (2, i, 0)),
        ],
        out_specs=pl.BlockSpec((8, 5 * C), lambda i: (0, 0)),
        out_shape=jax.ShapeDtypeStruct((8, 5 * C), jnp.float32),
    )(a, smq, smq)


def _layer_x(a, m, sums, N):
    C = a.shape[1]
    kf = jnp.float32(_K)
    cnt = jnp.float32(N * _K)
    sa = sums[:, 0:C]
    ss = sums[:, C:2 * C]
    sq = sums[:, 2 * C:3 * C]
    saa = sums[:, 3 * C:4 * C]
    sas = sums[:, 4 * C:5 * C]
    mean = (kf * sa + ss) / cnt
    e2 = (kf * saa + 2.0 * sas + sq) / cnt
    var = e2 - mean * mean
    rstd = jax.lax.rsqrt(var + 1e-5)
    x = (a + m - mean) * rstd
    return jnp.where(x >= 0, x, 0.2 * x)


def _apply1_body(a_ref, m_ref, sums_ref, w_ref, x1_ref, a2_ref, b2_ref, *, N):
    C = a_ref.shape[1]
    x = _layer_x(a_ref[:, :], m_ref[0, :, :], sums_ref[0:1, :], N)
    x1_ref[:, :] = x
    c2 = a2_ref.shape[1]
    a2_ref[:, :] = _dot(x, w_ref[:, pl.ds(0, c2)])
    b2_ref[:, :] = _dot(x, w_ref[:, pl.ds(c2, c2)])


def _apply1(a1, smq1, sums1, wcat2, N, tm=1024):
    NP, C = a1.shape
    c2 = wcat2.shape[1] // 2
    body = functools.partial(_apply1_body, N=N)
    return pl.pallas_call(
        body,
        grid=(NP // tm,),
        in_specs=[
            pl.BlockSpec((tm, C), lambda i: (i, 0)),
            pl.BlockSpec((1, tm, C), lambda i: (1, i, 0)),
            pl.BlockSpec((8, 5 * C), lambda i: (0, 0)),
            pl.BlockSpec((C, 2 * c2), lambda i: (0, 0)),
        ],
        out_specs=[
            pl.BlockSpec((tm, C), lambda i: (i, 0)),
            pl.BlockSpec((tm, c2), lambda i: (i, 0)),
            pl.BlockSpec((tm, c2), lambda i: (i, 0)),
        ],
        out_shape=[
            jax.ShapeDtypeStruct((NP, C), jnp.float32),
            jax.ShapeDtypeStruct((NP, c2), jnp.float32),
            jax.ShapeDtypeStruct((NP, c2), jnp.float32),
        ],
    )(a1, smq1, sums1, wcat2)


def _apply2_body(a_ref, m_ref, sums_ref, f_ref, x1_ref, w3_ref,
                 y_ref, ys_ref, *, N, tm):
    i = pl.program_id(0)
    C2 = a_ref.shape[1]
    C = f_ref.shape[1]
    x2 = _layer_x(a_ref[:, :], m_ref[0, :, :], sums_ref[0:1, :], N)
    y = (_dot(f_ref[:, :], w3_ref[pl.ds(0, C), :]) +
         _dot(x1_ref[:, :], w3_ref[pl.ds(C, C), :]) +
         _dot(x2, w3_ref[pl.ds(2 * C, 2 * C), :]))
    y_ref[:, :] = y
    row = i * tm + jax.lax.broadcasted_iota(jnp.int32, (tm, 1), 0)
    msk = (row < N).astype(_F32)
    ym = y * msk
    part = jnp.concatenate([
        jnp.sum(ym, 0, keepdims=True),
        jnp.sum(ym * y, 0, keepdims=True),
    ], axis=1)

    @pl.when(i == 0)
    def _():
        ys_ref[:, :] = jnp.zeros_like(ys_ref)

    ys_ref[0:1, :] += part


def _apply2(a2, smq2, sums2, featst, x1t, w3t, N, tm=1024):
    NP, C2 = a2.shape
    C = featst.shape[1]
    body = functools.partial(_apply2_body, N=N, tm=tm)
    return pl.pallas_call(
        body,
        grid=(NP // tm,),
        in_specs=[
            pl.BlockSpec((tm, C2), lambda i: (i, 0)),
            pl.BlockSpec((1, tm, C2), lambda i: (1, i, 0)),
            pl.BlockSpec((8, 5 * C2), lambda i: (0, 0)),
            pl.BlockSpec((tm, C), lambda i: (i, 0)),
            pl.BlockSpec((tm, C), lambda i: (i, 0)),
            pl.BlockSpec((4 * C, C), lambda i: (0, 0)),
        ],
        out_specs=[
            pl.BlockSpec((tm, C), lambda i: (i, 0)),
            pl.BlockSpec((8, 2 * C), lambda i: (0, 0)),
        ],
        out_shape=[
            jax.ShapeDtypeStruct((NP, C), jnp.float32),
            jax.ShapeDtypeStruct((8, 2 * C), jnp.float32),
        ],
    )(a2, smq2, sums2, featst, x1t, w3t)


def _norm_body(y_ref, ys_ref, o_ref, *, N):
    C = y_ref.shape[1]
    nf = jnp.float32(N)
    s = ys_ref[0:1, 0:C]
    s2 = ys_ref[0:1, C:2 * C]
    mean = s / nf
    var = s2 / nf - mean * mean
    rstd = jax.lax.rsqrt(var + 1e-5)
    x = (y_ref[:, :] - mean) * rstd
    o_ref[:, :] = jnp.where(x >= 0, x, 0.2 * x)


def _norm_final(y, ysums, N, tm=1024):
    NP, C = y.shape
    body = functools.partial(_norm_body, N=N)
    return pl.pallas_call(
        body,
        grid=(NP // tm,),
        in_specs=[
            pl.BlockSpec((tm, C), lambda i: (i, 0)),
            pl.BlockSpec((8, 2 * C), lambda i: (0, 0)),
        ],
        out_specs=pl.BlockSpec((tm, C), lambda i: (i, 0)),
        out_shape=jax.ShapeDtypeStruct((NP, C), jnp.float32),
    )(y, ysums)


def kernel(coords, features, W1, W2, W3):
    B, C, N = features.shape
    NP = ((N + 2559) // 2560) * 2560

    idx_full = _knn_pallas(coords, NP)
    idxf = idx_full[:, 1:1 + _K].reshape(-1)

    featst = jnp.zeros((NP, C), _F32).at[:N].set(features[0].T)
    W1a, W1b = W1[:, :C], W1[:, C:]
    wcat1 = jnp.concatenate([(W1a - W1b).T, W1b.T], axis=1)
    W2a, W2b = W2[:, :C], W2[:, C:]
    wcat2 = jnp.concatenate([(W2a - W2b).T, W2b.T], axis=1)
    w3t = W3.T

    a1, b1 = _matmul(featst, wcat1, (C, C))
    smq1 = _gather_reduce(b1, idxf)
    sums1 = _stats(a1, smq1, N)
    x1t, a2, b2 = _apply1(a1, smq1, sums1, wcat2, N)
    smq2 = _gather_reduce(b2, idxf)
    sums2 = _stats(a2, smq2, N)
    y, ysums = _apply2(a2, smq2, sums2, featst, x1t, w3t, N)
    out = _norm_final(y, ysums, N)
    return out[:N].T[None]


# speedup vs baseline: 9.0884x; 1.3567x over previous
import functools
import jax
import jax.numpy as jnp
from jax import lax
from jax.experimental import pallas as pl
from jax.experimental.pallas import tpu as pltpu
from jax.experimental.pallas import tpu_sc as plsc

_K = 10
_F32 = jnp.float32
_HI = jax.lax.Precision.HIGHEST


def _dot(a, b):
    return jax.lax.dot_general(a, b, (((1,), (0,)), ((), ())),
                               preferred_element_type=_F32, precision=_HI)


_FW = 16


def _dist_body(ct_ref, c_ref, d_ref, cm_ref):
    TM = ct_ref.shape[0]
    NP = c_ref.shape[1]
    CH = 256
    NCH = NP // CH

    xi = ct_ref[:, 0:1]
    yi = ct_ref[:, 1:2]
    zi = ct_ref[:, 2:3]
    sqi = xi * xi + yi * yi + zi * zi
    xib = xi.astype(jnp.bfloat16).astype(_F32)
    yib = yi.astype(jnp.bfloat16).astype(_F32)
    zib = zi.astype(jnp.bfloat16).astype(_F32)

    def pass1(c, carry):
        sl = pl.ds(c * CH, CH)
        xj = c_ref[0:1, sl]
        yj = c_ref[1:2, sl]
        zj = c_ref[2:3, sl]
        sqj = xj * xj + yj * yj + zj * zj
        xjb = xj.astype(jnp.bfloat16).astype(_F32)
        yjb = yj.astype(jnp.bfloat16).astype(_F32)
        zjb = zj.astype(jnp.bfloat16).astype(_F32)
        d = sqi + sqj - 2.0 * (xib * xjb + yib * yjb + zib * zjb)
        d_ref[2 * c, :, :] = d[:, 0:128]
        d_ref[2 * c + 1, :, :] = d[:, 128:256]
        cm = jnp.min(d.reshape(TM, CH // _FW, _FW), axis=2)
        cm_ref[:, c, :] = cm
        return carry

    jax.lax.fori_loop(0, NCH, pass1, 0)


def _csel_body(cm_ref, out_ref, scr, *, nsel):
    TM = cm_ref.shape[0]
    NF = cm_ref.shape[1]
    CH = 128
    NCH = NF // CH
    INF = jnp.float32(3e38)
    BIGI = jnp.int32(2**30)

    scr[:, :] = cm_ref[:, :]
    lane_out = jax.lax.broadcasted_iota(jnp.int32, (TM, 128), 1)

    def ext_body(t, carry):
        acc, prev_idx = carry

        def chunk_body(c, ch_carry):
            bval, bidx = ch_carry
            sl = pl.ds(c * CH, CH)
            d = scr[:, sl]
            gcol = jax.lax.broadcasted_iota(jnp.int32, (TM, CH), 1) + c * CH
            dm = jnp.where(gcol == prev_idx, INF, d)
            scr[:, sl] = dm
            cmin = jnp.min(dm, axis=1, keepdims=True)
            cidx = jnp.min(jnp.where(dm == cmin, gcol, BIGI), axis=1,
                           keepdims=True)
            better = cmin < bval
            return (jnp.where(better, cmin, bval),
                    jnp.where(better, cidx, bidx))

        bval0 = jnp.full((TM, 1), INF, jnp.float32)
        bidx0 = jnp.full((TM, 1), BIGI, jnp.int32)
        _, bidx = jax.lax.fori_loop(0, NCH, chunk_body, (bval0, bidx0))
        acc = jnp.where(lane_out == t, bidx, acc)
        return acc, bidx

    acc0 = jnp.zeros((TM, 128), jnp.int32)
    prev0 = jnp.full((TM, 1), -1, jnp.int32)
    acc, _ = jax.lax.fori_loop(0, nsel, ext_body, (acc0, prev0))
    out_ref[:, :] = acc


def _allmin16(v):
    i16 = lax.iota(jnp.int32, 16)
    for s in (8, 4, 2, 1):
        v = jnp.minimum(v, v[(i16 + s) & 15])
    return v


def _scfinal_body(dflat_hbm, c16_hbm, knn_hbm, cid_v, cand_v, val_v, col_v,
                  out_v, sem, *, PPW, NP):
    NSEL = 16
    INF = jnp.float32(3e38)
    BIGI = jnp.int32(2**30)
    wid = lax.axis_index("s") * 2 + lax.axis_index("c")
    base = wid * PPW
    iota16 = lax.iota(jnp.int32, 16)

    def row(r, carry):
        rg = base + r
        pltpu.sync_copy(c16_hbm.at[rg, pl.ds(0, NSEL)], cid_v)
        cidv = cid_v[...]
        gidx = (cidv >> 3) * NP + rg
        pltpu.async_copy(dflat_hbm.at[gidx], cand_v, sem).wait()
        one = jnp.int32(1)
        for j in range(NSEL):
            cj = cidv[jnp.full((16,), j, jnp.int32)]
            col_v[j, :] = cj * _FW + iota16
            sub = cj & 7
            v = jnp.zeros((16,), jnp.float32)
            for h in range(8):
                ef = (one - jnp.minimum(jnp.abs(sub - h), one)
                      ).astype(jnp.float32)
                v = v + cand_v[j, pl.ds(h * 16, 16)] * ef
            val_v[j, :] = v

        def extract(t, tcarry):
            acc, pv, pi = tcarry

            def fold(j, m):
                v = val_v[j, :]
                c = col_v[j, :]
                e = (v > pv) | ((v == pv) & (c > pi))
                mv = jnp.where(e, v, INF)
                val_v[j, :] = mv
                return jnp.minimum(m, mv)

            m0 = jnp.full((16,), INF, jnp.float32)
            ms = _allmin16(jax.lax.fori_loop(0, NSEL, fold, m0))

            def locate(j, ci):
                eq = val_v[j, :] == ms
                return jnp.minimum(ci, jnp.where(eq, col_v[j, :], BIGI))

            ci0 = jnp.full((16,), BIGI, jnp.int32)
            cis = _allmin16(jax.lax.fori_loop(0, NSEL, locate, ci0))
            acc = jnp.where(iota16 == t, cis, acc)
            return acc, ms, cis

        acc0 = jnp.zeros((16,), jnp.int32)
        pv0 = jnp.full((16,), -INF, jnp.float32)
        pi0 = jnp.full((16,), -1, jnp.int32)
        acc, _, _ = jax.lax.fori_loop(0, _K + 1, extract, (acc0, pv0, pi0))
        out_v[...] = acc
        pltpu.sync_copy(out_v, knn_hbm.at[rg])
        return carry

    jax.lax.fori_loop(0, PPW, row, 0)


def _knn_idx16(coords, NP, tm=256):
    _, _, N = coords.shape
    NF = NP // _FW
    c = coords[0]
    c_pad = jnp.full((8, NP), 1e6, jnp.float32)
    c_pad = c_pad.at[:3, :N].set(c)
    c_pad = c_pad.at[3:, :].set(0.0)
    ct_pad = c_pad.T

    d, cm = pl.pallas_call(
        _dist_body,
        grid=(NP // tm,),
        in_specs=[
            pl.BlockSpec((tm, 8), lambda i: (i, 0)),
            pl.BlockSpec((8, NP), lambda i: (0, 0)),
        ],
        out_specs=[
            pl.BlockSpec((NP // 128, tm, 128), lambda i: (0, i, 0)),
            pl.BlockSpec((tm, NP // 256, 16), lambda i: (i, 0, 0)),
        ],
        out_shape=[
            jax.ShapeDtypeStruct((NP // 128, NP, 128), jnp.float32),
            jax.ShapeDtypeStruct((NP, NP // 256, 16), jnp.float32),
        ],
    )(ct_pad, c_pad)
    cm = cm.reshape(NP, NF)

    c16 = pl.pallas_call(
        functools.partial(_csel_body, nsel=16),
        grid=(NP // tm,),
        in_specs=[pl.BlockSpec((tm, NF), lambda i: (i, 0))],
        out_specs=pl.BlockSpec((tm, 128), lambda i: (i, 0)),
        out_shape=jax.ShapeDtypeStruct((NP, 128), jnp.int32),
        scratch_shapes=[pltpu.VMEM((tm, NF), jnp.float32)],
    )(cm)

    NW = 32
    PPW = NP // NW
    dflat = d.reshape((NP // 128) * NP, 128)
    mesh = plsc.VectorSubcoreMesh(core_axis_name="c", subcore_axis_name="s")
    body = functools.partial(_scfinal_body, PPW=PPW, NP=NP)
    knn = functools.partial(
        pl.kernel, mesh=mesh,
        out_type=jax.ShapeDtypeStruct((NP, 16), jnp.int32),
        scratch_types=[
            pltpu.VMEM((16,), jnp.int32),
            pltpu.VMEM((16, 128), jnp.float32),
            pltpu.VMEM((16, 16), jnp.float32),
            pltpu.VMEM((16, 16), jnp.int32),
            pltpu.VMEM((16,), jnp.int32),
            pltpu.SemaphoreType.DMA,
        ])(body)(dflat, c16)
    return knn


def _knn_body(ct_ref, c_ref, out_ref, d_scr):
    TM = ct_ref.shape[0]
    NP = c_ref.shape[1]
    CH = 256
    NCH = NP // CH
    INF = jnp.float32(3e38)
    BIGI = jnp.int32(2**30)

    xi = ct_ref[:, 0:1]
    yi = ct_ref[:, 1:2]
    zi = ct_ref[:, 2:3]
    sqi = xi * xi + yi * yi + zi * zi
    xib = xi.astype(jnp.bfloat16).astype(_F32)
    yib = yi.astype(jnp.bfloat16).astype(_F32)
    zib = zi.astype(jnp.bfloat16).astype(_F32)

    def pass1(c, carry):
        sl = pl.ds(c * CH, CH)
        xj = c_ref[0:1, sl]
        yj = c_ref[1:2, sl]
        zj = c_ref[2:3, sl]
        sqj = xj * xj + yj * yj + zj * zj
        xjb = xj.astype(jnp.bfloat16).astype(_F32)
        yjb = yj.astype(jnp.bfloat16).astype(_F32)
        zjb = zj.astype(jnp.bfloat16).astype(_F32)
        d = sqi + sqj - 2.0 * (xib * xjb + yib * yjb + zib * zjb)
        d_scr[:, sl] = d
        return carry

    jax.lax.fori_loop(0, NCH, pass1, 0)

    lane_out = jax.lax.broadcasted_iota(jnp.int32, (TM, 128), 1)

    def ext_body(t, carry):
        acc, prev_idx = carry

        def chunk_body(c, ch_carry):
            bval, bidx = ch_carry
            sl = pl.ds(c * CH, CH)
            d = d_scr[:, sl]
            gcol = jax.lax.broadcasted_iota(jnp.int32, (TM, CH), 1) + c * CH
            dm = jnp.where(gcol == prev_idx, INF, d)
            d_scr[:, sl] = dm
            cmin = jnp.min(dm, axis=1, keepdims=True)
            cidx = jnp.min(jnp.where(dm == cmin, gcol, BIGI), axis=1,
                           keepdims=True)
            better = cmin < bval
            return (jnp.where(better, cmin, bval),
                    jnp.where(better, cidx, bidx))

        bval0 = jnp.full((TM, 1), INF, jnp.float32)
        bidx0 = jnp.full((TM, 1), BIGI, jnp.int32)
        _, bidx = jax.lax.fori_loop(0, NCH, chunk_body, (bval0, bidx0))
        acc = jnp.where(lane_out == t, bidx, acc)
        return acc, bidx

    acc0 = jnp.zeros((TM, 128), jnp.int32)
    prev0 = jnp.full((TM, 1), -1, jnp.int32)
    acc, _ = jax.lax.fori_loop(0, _K + 1, ext_body, (acc0, prev0))
    out_ref[:, :] = acc


def _knn_pallas(coords, NP, tm=256):
    _, _, N = coords.shape
    c = coords[0]
    c_pad = jnp.full((8, NP), 1e6, jnp.float32)
    c_pad = c_pad.at[:3, :N].set(c)
    c_pad = c_pad.at[3:, :].set(0.0)
    ct_pad = c_pad.T

    return pl.pallas_call(
        _knn_body,
        grid=(NP // tm,),
        in_specs=[
            pl.BlockSpec((tm, 8), lambda i: (i, 0)),
            pl.BlockSpec((8, NP), lambda i: (0, 0)),
        ],
        out_specs=pl.BlockSpec((tm, 128), lambda i: (i, 0)),
        out_shape=jax.ShapeDtypeStruct((NP, 128), jnp.int32),
        scratch_shapes=[pltpu.VMEM((tm, NP), jnp.float32)],
    )(ct_pad, c_pad)


def _gr_body(bt_hbm, idx_hbm, out_hbm, idx_v, rows_v, s_v, m_v, q_v, sem,
             *, PPW, CP, GP, C):
    NF = CP // GP
    NCHK = PPW // CP
    wid = lax.axis_index("s") * 2 + lax.axis_index("c")
    base = wid * PPW

    def chunk(ch, carry):
        pbase = base + ch * CP
        pltpu.sync_copy(idx_hbm.at[pl.ds(pbase * _K, CP * _K)], idx_v)
        cps = [pltpu.async_copy(
                   bt_hbm.at[idx_v.at[pl.ds(f * GP * _K, GP * _K)]],
                   rows_v.at[pl.ds(f * GP * _K, GP * _K)], sem)
               for f in range(NF)]
        for cp_ in cps:
            cp_.wait()

        def point(p, c2):
            for c in range(C // 16):
                sl = pl.ds(c * 16, 16)
                r = rows_v[p * _K, sl]
                s = r
                m = r
                q = r * r
                for j in range(1, _K):
                    r = rows_v[p * _K + j, sl]
                    s = s + r
                    m = jnp.maximum(m, r)
                    q = q + r * r
                s_v[p, sl] = s
                m_v[p, sl] = m
                q_v[p, sl] = q
            return c2

        jax.lax.fori_loop(0, CP, point, 0)
        pltpu.sync_copy(s_v, out_hbm.at[0, pl.ds(pbase, CP)])
        pltpu.sync_copy(m_v, out_hbm.at[1, pl.ds(pbase, CP)])
        pltpu.sync_copy(q_v, out_hbm.at[2, pl.ds(pbase, CP)])
        return carry

    jax.lax.fori_loop(0, NCHK, chunk, 0)


def _gather_reduce(bt, idxf):
    NP, C = bt.shape
    NW = 32
    PPW = NP // NW
    CP = 16
    GP = 8
    mesh = plsc.VectorSubcoreMesh(core_axis_name="c", subcore_axis_name="s")
    body = functools.partial(_gr_body, PPW=PPW, CP=CP, GP=GP, C=C)
    f = functools.partial(
        pl.kernel, mesh=mesh,
        out_type=jax.ShapeDtypeStruct((3, NP, C), jnp.float32),
        scratch_types=[
            pltpu.VMEM((CP * _K,), jnp.int32),
            pltpu.VMEM((CP * _K, C), jnp.float32),
            pltpu.VMEM((CP, C), jnp.float32),
            pltpu.VMEM((CP, C), jnp.float32),
            pltpu.VMEM((CP, C), jnp.float32),
            pltpu.SemaphoreType.DMA,
        ])(body)
    return f(bt, idxf)


def _mm_body(x_ref, w_ref, *out_refs):
    x = x_ref[:, :]
    off = 0
    for o in out_refs:
        co = o.shape[1]
        o[:, :] = _dot(x, w_ref[:, pl.ds(off, co)])
        off += co


def _matmul(x, w, splits, tm=1024):
    NP, cin = x.shape
    cout = w.shape[1]
    return pl.pallas_call(
        _mm_body,
        grid=(NP // tm,),
        in_specs=[
            pl.BlockSpec((tm, cin), lambda i: (i, 0)),
            pl.BlockSpec((cin, cout), lambda i: (0, 0)),
        ],
        out_specs=[pl.BlockSpec((tm, co), lambda i: (i, 0)) for co in splits],
        out_shape=[jax.ShapeDtypeStruct((NP, co), jnp.float32)
                   for co in splits],
    )(x, w)


def _stats_body(a_ref, s_ref, q_ref, o_ref, *, N, tm):
    i = pl.program_id(0)
    C = a_ref.shape[1]
    row = i * tm + jax.lax.broadcasted_iota(jnp.int32, (tm, 1), 0)
    msk = (row < N).astype(_F32)
    a = a_ref[:, :] * msk
    s = s_ref[0, :, :] * msk
    q = q_ref[0, :, :] * msk
    part = jnp.concatenate([
        jnp.sum(a, 0, keepdims=True),
        jnp.sum(s, 0, keepdims=True),
        jnp.sum(q, 0, keepdims=True),
        jnp.sum(a * a, 0, keepdims=True),
        jnp.sum(a * s, 0, keepdims=True),
    ], axis=1)

    @pl.when(i == 0)
    def _():
        o_ref[:, :] = jnp.zeros_like(o_ref)

    o_ref[0:1, :] += part


def _stats(a, smq, N, tm=1024):
    NP, C = a.shape
    body = functools.partial(_stats_body, N=N, tm=tm)
    return pl.pallas_call(
        body,
        grid=(NP // tm,),
        in_specs=[
            pl.BlockSpec((tm, C), lambda i: (i, 0)),
            pl.BlockSpec((1, tm, C), lambda i: (0, i, 0)),
            pl.BlockSpec((1, tm, C), lambda i: (2, i, 0)),
        ],
        out_specs=pl.BlockSpec((8, 5 * C), lambda i: (0, 0)),
        out_shape=jax.ShapeDtypeStruct((8, 5 * C), jnp.float32),
    )(a, smq, smq)


def _layer_x(a, m, sums, N):
    C = a.shape[1]
    kf = jnp.float32(_K)
    cnt = jnp.float32(N * _K)
    sa = sums[:, 0:C]
    ss = sums[:, C:2 * C]
    sq = sums[:, 2 * C:3 * C]
    saa = sums[:, 3 * C:4 * C]
    sas = sums[:, 4 * C:5 * C]
    mean = (kf * sa + ss) / cnt
    e2 = (kf * saa + 2.0 * sas + sq) / cnt
    var = e2 - mean * mean
    rstd = jax.lax.rsqrt(var + 1e-5)
    x = (a + m - mean) * rstd
    return jnp.where(x >= 0, x, 0.2 * x)


def _apply1_body(a_ref, m_ref, sums_ref, w_ref, x1_ref, a2_ref, b2_ref, *, N):
    C = a_ref.shape[1]
    x = _layer_x(a_ref[:, :], m_ref[0, :, :], sums_ref[0:1, :], N)
    x1_ref[:, :] = x
    c2 = a2_ref.shape[1]
    a2_ref[:, :] = _dot(x, w_ref[:, pl.ds(0, c2)])
    b2_ref[:, :] = _dot(x, w_ref[:, pl.ds(c2, c2)])


def _apply1(a1, smq1, sums1, wcat2, N, tm=1024):
    NP, C = a1.shape
    c2 = wcat2.shape[1] // 2
    body = functools.partial(_apply1_body, N=N)
    return pl.pallas_call(
        body,
        grid=(NP // tm,),
        in_specs=[
            pl.BlockSpec((tm, C), lambda i: (i, 0)),
            pl.BlockSpec((1, tm, C), lambda i: (1, i, 0)),
            pl.BlockSpec((8, 5 * C), lambda i: (0, 0)),
            pl.BlockSpec((C, 2 * c2), lambda i: (0, 0)),
        ],
        out_specs=[
            pl.BlockSpec((tm, C), lambda i: (i, 0)),
            pl.BlockSpec((tm, c2), lambda i: (i, 0)),
            pl.BlockSpec((tm, c2), lambda i: (i, 0)),
        ],
        out_shape=[
            jax.ShapeDtypeStruct((NP, C), jnp.float32),
            jax.ShapeDtypeStruct((NP, c2), jnp.float32),
            jax.ShapeDtypeStruct((NP, c2), jnp.float32),
        ],
    )(a1, smq1, sums1, wcat2)


def _apply2_body(a_ref, m_ref, sums_ref, f_ref, x1_ref, w3_ref,
                 y_ref, ys_ref, *, N, tm):
    i = pl.program_id(0)
    C2 = a_ref.shape[1]
    C = f_ref.shape[1]
    x2 = _layer_x(a_ref[:, :], m_ref[0, :, :], sums_ref[0:1, :], N)
    y = (_dot(f_ref[:, :], w3_ref[pl.ds(0, C), :]) +
         _dot(x1_ref[:, :], w3_ref[pl.ds(C, C), :]) +
         _dot(x2, w3_ref[pl.ds(2 * C, 2 * C), :]))
    y_ref[:, :] = y
    row = i * tm + jax.lax.broadcasted_iota(jnp.int32, (tm, 1), 0)
    msk = (row < N).astype(_F32)
    ym = y * msk
    part = jnp.concatenate([
        jnp.sum(ym, 0, keepdims=True),
        jnp.sum(ym * y, 0, keepdims=True),
    ], axis=1)

    @pl.when(i == 0)
    def _():
        ys_ref[:, :] = jnp.zeros_like(ys_ref)

    ys_ref[0:1, :] += part


def _apply2(a2, smq2, sums2, featst, x1t, w3t, N, tm=1024):
    NP, C2 = a2.shape
    C = featst.shape[1]
    body = functools.partial(_apply2_body, N=N, tm=tm)
    return pl.pallas_call(
        body,
        grid=(NP // tm,),
        in_specs=[
            pl.BlockSpec((tm, C2), lambda i: (i, 0)),
            pl.BlockSpec((1, tm, C2), lambda i: (1, i, 0)),
            pl.BlockSpec((8, 5 * C2), lambda i: (0, 0)),
            pl.BlockSpec((tm, C), lambda i: (i, 0)),
            pl.BlockSpec((tm, C), lambda i: (i, 0)),
            pl.BlockSpec((4 * C, C), lambda i: (0, 0)),
        ],
        out_specs=[
            pl.BlockSpec((tm, C), lambda i: (i, 0)),
            pl.BlockSpec((8, 2 * C), lambda i: (0, 0)),
        ],
        out_shape=[
            jax.ShapeDtypeStruct((NP, C), jnp.float32),
            jax.ShapeDtypeStruct((8, 2 * C), jnp.float32),
        ],
    )(a2, smq2, sums2, featst, x1t, w3t)


def _norm_body(y_ref, ys_ref, o_ref, *, N):
    C = y_ref.shape[1]
    nf = jnp.float32(N)
    s = ys_ref[0:1, 0:C]
    s2 = ys_ref[0:1, C:2 * C]
    mean = s / nf
    var = s2 / nf - mean * mean
    rstd = jax.lax.rsqrt(var + 1e-5)
    x = (y_ref[:, :] - mean) * rstd
    o_ref[:, :] = jnp.where(x >= 0, x, 0.2 * x)


def _norm_final(y, ysums, N, tm=1024):
    NP, C = y.shape
    body = functools.partial(_norm_body, N=N)
    return pl.pallas_call(
        body,
        grid=(NP // tm,),
        in_specs=[
            pl.BlockSpec((tm, C), lambda i: (i, 0)),
            pl.BlockSpec((8, 2 * C), lambda i: (0, 0)),
        ],
        out_specs=pl.BlockSpec((tm, C), lambda i: (i, 0)),
        out_shape=jax.ShapeDtypeStruct((NP, C), jnp.float32),
    )(y, ysums)


def kernel(coords, features, W1, W2, W3):
    B, C, N = features.shape
    NP = ((N + 2559) // 2560) * 2560

    knn16 = _knn_idx16(coords, NP)
    idxf = knn16[:, 1:1 + _K].reshape(-1)

    featst = jnp.zeros((NP, C), _F32).at[:N].set(features[0].T)
    W1a, W1b = W1[:, :C], W1[:, C:]
    wcat1 = jnp.concatenate([(W1a - W1b).T, W1b.T], axis=1)
    W2a, W2b = W2[:, :C], W2[:, C:]
    wcat2 = jnp.concatenate([(W2a - W2b).T, W2b.T], axis=1)
    w3t = W3.T

    a1, b1 = _matmul(featst, wcat1, (C, C))
    smq1 = _gather_reduce(b1, idxf)
    sums1 = _stats(a1, smq1, N)
    x1t, a2, b2 = _apply1(a1, smq1, sums1, wcat2, N)
    smq2 = _gather_reduce(b2, idxf)
    sums2 = _stats(a2, smq2, N)
    y, ysums = _apply2(a2, smq2, sums2, featst, x1t, w3t, N)
    out = _norm_final(y, ysums, N)
    return out[:N].T[None]
